# Initial kernel scaffold; baseline (speedup 1.0000x reference)
import jax
import jax.numpy as jnp
from jax.experimental import pallas as pl


def kernel(yi, Z, Rij, idx_i, idx_j, idx_m, c6_ref, r4r2):
    raise NotImplementedError("write your pallas kernel here")


# speedup vs baseline: 32.0557x; 32.0557x over previous
import functools

import jax
import jax.numpy as jnp
from jax import lax
from jax.experimental import pallas as pl
from jax.experimental.pallas import tpu as pltpu
from jax.experimental.pallas import tpu_sc as plsc

KE = 14.399645
S6, S8, A1, A2 = 1.0, 1.2, 0.4, 5.0
CUTON = 2.5
SW_INV = 0.2
LR = 10.0
SQRT3 = 1.7320508075688772

NC, NS, NW, L = 2, 16, 32, 16


def _rsqrt(x):
    b = lax.bitcast_convert_type(x, jnp.int32)
    y = lax.bitcast_convert_type(
        jnp.int32(0x5F3759DF) - lax.shift_right_arithmetic(b, 1), jnp.float32)
    for _ in range(3):
        y = y * (1.5 - 0.5 * x * y * y)
    return y


def _make_atom_kernel(n_pad):
    nps = n_pad // NS
    npa = n_pad // NW

    @functools.partial(
        pl.kernel,
        out_type=jax.ShapeDtypeStruct((n_pad, 8), jnp.float32),
        mesh=plsc.VectorSubcoreMesh(core_axis_name="c", subcore_axis_name="s"),
        compiler_params=pltpu.CompilerParams(needs_layout_passes=False, use_tc_tiling_on_sc=False),
        scratch_types=[
            pltpu.VMEM((1024,), jnp.float32),
            pltpu.VMEM((nps,), jnp.int32),
            pltpu.VMEM((nps,), jnp.float32),
            pltpu.VMEM((96,), jnp.float32),
            pltpu.VMEM((96,), jnp.float32),
            pltpu.VMEM((8, 128), jnp.int32),
            pltpu.VMEM((1024,), jnp.float32),
            pltpu.VMEM((32,), jnp.float32),
            pltpu.VMEM((npa,), jnp.int32),
            pltpu.VMEM((npa,), jnp.float32),
            pltpu.VMEM((npa,), jnp.int32),
            pltpu.VMEM((npa, 8), jnp.float32),
            pltpu.VMEM_SHARED((1024,), jnp.float32),
        ],
    )
    def atom_kernel(charge_hbm, idxm_hbm, z_hbm, sqc6_hbm, sqr4_hbm, idx8_hbm,
                    tab_hbm, acc_v, idxm1_v, chg1_v, sqc6_v, sqr4_v, idx8_v,
                    binsf_v, qdiv_v, idxm2_v, chg2_v, z2_v, tab_v, bins_sh):
        c = lax.axis_index("c")
        s = lax.axis_index("s")
        wid = s * NC + c
        iota = lax.iota(jnp.int32, 16)
        zero16 = jnp.zeros((16,), jnp.float32)

        pltpu.sync_copy(idxm_hbm.at[pl.ds(s * nps, nps)], idxm1_v)
        pltpu.sync_copy(charge_hbm.at[pl.ds(s * nps, nps)], chg1_v)
        pltpu.sync_copy(idx8_hbm, idx8_v)
        pltpu.sync_copy(sqc6_hbm, sqc6_v)
        pltpu.sync_copy(sqr4_hbm, sqr4_v)

        def zero_body(i, _):
            acc_v[pl.ds(i * 16, 16)] = zero16
            return 0
        lax.fori_loop(0, 64, zero_body, 0)

        @pl.when(s == 0)
        def _():
            pltpu.sync_copy(acc_v, bins_sh)
        plsc.subcore_barrier()

        m_lo = idxm1_v[pl.ds(0, 16)][0]
        m_hi = idxm1_v[pl.ds(nps - 16, 16)][15]

        def m_body(m, _):
            def v_body(v, carry):
                qa_, ca_ = carry
                im = idxm1_v[pl.ds(v * 16, 16)]
                ch = chg1_v[pl.ds(v * 16, 16)]
                mask = im == m
                qa_ = qa_ + jnp.where(mask, -ch, 0.0)
                ca_ = ca_ + jnp.where(mask, 1.0, 0.0)
                return (qa_, ca_)
            qa_, ca_ = lax.fori_loop(0, nps // 16, v_body, (zero16, zero16))
            plsc.addupdate(acc_v.at[pl.ds(m * 16, 16)], qa_)
            plsc.addupdate(acc_v.at[pl.ds(512 + m * 16, 16)], ca_)
            return 0
        lax.fori_loop(m_lo, m_hi + 1, m_body, 0)

        for k in range(8):
            pltpu.sync_copy(acc_v.at[pl.ds(k * 128, 128)],
                            bins_sh.at[idx8_v.at[k]], add=True)
        plsc.subcore_barrier()

        pltpu.sync_copy(bins_sh, binsf_v)
        qlo = qhi = clo = chi = zero16
        for k in range(16):
            qlo = qlo + plsc.load_gather(binsf_v, [iota * 16 + k])
            qhi = qhi + plsc.load_gather(binsf_v, [(iota + 16) * 16 + k])
            clo = clo + plsc.load_gather(binsf_v, [512 + iota * 16 + k])
            chi = chi + plsc.load_gather(binsf_v, [512 + (iota + 16) * 16 + k])
        qdiv_v[pl.ds(0, 16)] = qlo / jnp.maximum(clo, 1.0)
        qdiv_v[pl.ds(16, 16)] = qhi / jnp.maximum(chi, 1.0)

        base = wid * npa
        pltpu.sync_copy(idxm_hbm.at[pl.ds(base, npa)], idxm2_v)
        pltpu.sync_copy(charge_hbm.at[pl.ds(base, npa)], chg2_v)
        pltpu.sync_copy(z_hbm.at[pl.ds(base, npa)], z2_v)

        def a_body(v, _):
            off = v * 16
            ch = chg2_v[pl.ds(off, 16)]
            zz = z2_v[pl.ds(off, 16)]
            mm = jnp.minimum(idxm2_v[pl.ds(off, 16)], 31)
            qa = ch + plsc.load_gather(qdiv_v, [mm])
            sc6 = (plsc.load_gather(sqc6_v, [zz])
                   * jnp.exp(1.5 * (1.0 - jnp.exp(2.0 * qa))))
            srz = plsc.load_gather(sqr4_v, [zz])
            row = iota + off
            plsc.store_scatter(tab_v, [row, jnp.full((16,), 0, jnp.int32)], qa)
            plsc.store_scatter(tab_v, [row, jnp.full((16,), 1, jnp.int32)], sc6)
            plsc.store_scatter(tab_v, [row, jnp.full((16,), 2, jnp.int32)], srz)
            plsc.store_scatter(tab_v, [row, jnp.full((16,), 3, jnp.int32)], zero16)
            return 0
        lax.fori_loop(0, npa // 16, a_body, 0)
        pltpu.sync_copy(tab_v, tab_hbm.at[pl.ds(base, npa), :])

    return atom_kernel


def _make_edge_kernel(n_pad, n_acc, e_total, ec):
    ecd = ec // 128
    nchunk = e_total // ec
    k_per_tile = (nchunk + NW - 1) // NW
    nsl = n_acc // NS

    @functools.partial(
        pl.kernel,
        out_type=jax.ShapeDtypeStruct((NC, n_acc), jnp.float32),
        mesh=plsc.VectorSubcoreMesh(core_axis_name="c", subcore_axis_name="s"),
        compiler_params=pltpu.CompilerParams(needs_layout_passes=False, use_tc_tiling_on_sc=False),
        scratch_types=[
            pltpu.VMEM((ec, 3), jnp.float32),
            pltpu.VMEM((ecd, 128), jnp.int32),
            pltpu.VMEM((ecd, 128), jnp.int32),
            pltpu.VMEM((ec, 8), jnp.float32),
            pltpu.VMEM((ec, 8), jnp.float32),
            pltpu.VMEM((ec,), jnp.float32),
            pltpu.VMEM((nsl,), jnp.float32),
            pltpu.VMEM_SHARED((n_acc,), jnp.float32),
            pltpu.SemaphoreType.DMA,
        ],
    )
    def edge_kernel(tab_hbm, rij_hbm, ii2_hbm, jj2_hbm, zeros_hbm, part_hbm,
                    rij_v, ii_v, jj_v, ri_v, rj_v, val_v, stage_v, acc_sh, sem):
        c = lax.axis_index("c")
        s = lax.axis_index("s")
        wid = s * NC + c
        iota = lax.iota(jnp.int32, 16)
        col0 = jnp.full((16,), 0, jnp.int32)
        col1 = jnp.full((16,), 1, jnp.int32)
        col2 = jnp.full((16,), 2, jnp.int32)

        @pl.when(s == 0)
        def _():
            pltpu.sync_copy(zeros_hbm, acc_sh)
        plsc.subcore_barrier()

        def chunk_body(k, _):
            cid = k * NW + wid

            @pl.when(cid < nchunk)
            def _():
                eb = cid * ec
                pltpu.sync_copy(rij_hbm.at[pl.ds(eb, ec)], rij_v)
                pltpu.sync_copy(ii2_hbm.at[pl.ds(cid * ecd, ecd)], ii_v)
                pltpu.sync_copy(jj2_hbm.at[pl.ds(cid * ecd, ecd)], jj_v)
                cps = []
                for r in range(ecd):
                    cps.append(pltpu.async_copy(
                        tab_hbm.at[ii_v.at[r]],
                        ri_v.at[pl.ds(r * 128, 128)], sem))
                    cps.append(pltpu.async_copy(
                        tab_hbm.at[jj_v.at[r]],
                        rj_v.at[pl.ds(r * 128, 128)], sem))
                for cp in cps:
                    cp.wait()

                def v_body(v, _):
                    off = v * 16
                    row = iota + off
                    x = plsc.load_gather(rij_v, [row, col0])
                    y = plsc.load_gather(rij_v, [row, col1])
                    z = plsc.load_gather(rij_v, [row, col2])
                    qi = plsc.load_gather(ri_v, [row, col0])
                    ci = plsc.load_gather(ri_v, [row, col1])
                    si = plsc.load_gather(ri_v, [row, col2])
                    qj = plsc.load_gather(rj_v, [row, col0])
                    cj = plsc.load_gather(rj_v, [row, col1])
                    sj = plsc.load_gather(rj_v, [row, col2])

                    r2 = x * x + y * y + z * z
                    rs = _rsqrt(r2)
                    d = r2 * rs
                    r2s = r2 + 1.0
                    rss = _rsqrt(r2s)
                    dsh = r2s * rss
                    t = (d - CUTON) * SW_INV
                    poly = 1.0 + (t * t * t) * (-10.0 + t * (15.0 - 6.0 * t))
                    f = jnp.where(t <= 0.0, 1.0,
                                  jnp.where(t >= 1.0, 0.0, poly))
                    coul = jnp.where(d < LR, rs + d * 0.01 - 0.2, 0.0)
                    damp = jnp.where(dsh < LR, rss + dsh * 0.01 - 0.2, 0.0)
                    epair = ((KE * 0.5) * qi * qj
                             * (f * damp + (1.0 - f) * coul))
                    c6 = ci * cj
                    sqq = SQRT3 * si * sj
                    qq = sqq * sqq
                    r0 = A1 * sqq + A2
                    r02 = r0 * r0
                    r06 = r02 * r02 * r02
                    r08 = r06 * r02
                    r6 = r2 * r2 * r2
                    r8 = r6 * r2
                    dpair = -0.5 * (S6 * c6 / (r6 + r06)
                                    + S8 * c6 * qq / (r8 + r08))
                    val_v[pl.ds(off, 16)] = epair + dpair
                    return 0
                lax.fori_loop(0, ec // 16, v_body, 0)

                for r in range(ecd):
                    pltpu.sync_copy(val_v.at[pl.ds(r * 128, 128)],
                                    acc_sh.at[ii_v.at[r]], add=True)
            return 0
        lax.fori_loop(0, k_per_tile, chunk_body, 0)
        plsc.subcore_barrier()

        pltpu.sync_copy(acc_sh.at[pl.ds(s * nsl, nsl)], stage_v)
        pltpu.sync_copy(stage_v, part_hbm.at[c, pl.ds(s * nsl, nsl)])

    return edge_kernel


def _combine_body(p_ref, e_ref, o_ref):
    o_ref[...] = p_ref[0] + p_ref[1] + e_ref[...]


def kernel(yi, Z, Rij, idx_i, idx_j, idx_m, c6_ref, r4r2):
    n = Z.shape[0]
    e_total = idx_i.shape[0]
    f32, i32 = jnp.float32, jnp.int32

    n_pad = ((n + 512 - 1) // 512) * 512
    n_acc = ((n + 128 - 1) // 128) * 128
    ec = 2048

    energy = yi[:, 0]
    charge = yi[:, 1]

    charge_p = jnp.concatenate([charge, jnp.zeros((n_pad - n,), f32)])
    idxm_p = jnp.concatenate([idx_m, jnp.full((n_pad - n,), 32, i32)])
    z_p = jnp.concatenate([Z, jnp.zeros((n_pad - n,), i32)])
    sqc6 = jnp.sqrt(jnp.concatenate([c6_ref, jnp.ones((1,), f32)]))
    sqr4 = jnp.sqrt(jnp.concatenate([r4r2, jnp.ones((1,), f32)]))
    idx8 = jnp.arange(1024, dtype=i32).reshape(8, 128)

    tab = _make_atom_kernel(n_pad)(charge_p, idxm_p, z_p, sqc6, sqr4, idx8)

    ii2 = idx_i.reshape(e_total // 128, 128)
    jj2 = idx_j.reshape(e_total // 128, 128)
    zeros_acc = jnp.zeros((n_acc,), f32)
    part = _make_edge_kernel(n_pad, n_acc, e_total, ec)(
        tab, Rij, ii2, jj2, zeros_acc)

    en_p = jnp.concatenate([energy, jnp.zeros((n_acc - n,), f32)])
    rows = n_acc // 128
    out2 = pl.pallas_call(
        _combine_body,
        out_shape=jax.ShapeDtypeStruct((rows, 128), f32),
    )(part.reshape(NC, rows, 128), en_p.reshape(rows, 128))
    return out2.reshape(n_acc)[:n][:, None]


# speedup vs baseline: 34.2916x; 1.0698x over previous
import functools

import jax
import jax.numpy as jnp
from jax import lax
from jax.experimental import pallas as pl
from jax.experimental.pallas import tpu as pltpu
from jax.experimental.pallas import tpu_sc as plsc

KE = 14.399645
S6, S8, A1, A2 = 1.0, 1.2, 0.4, 5.0
CUTON = 2.5
SW_INV = 0.2
LR = 10.0
SQRT3 = 1.7320508075688772

NC, NS, NW, L = 2, 16, 32, 16


def _rsqrt(x):
    b = lax.bitcast_convert_type(x, jnp.int32)
    y = lax.bitcast_convert_type(
        jnp.int32(0x5F3759DF) - lax.shift_right_arithmetic(b, 1), jnp.float32)
    for _ in range(3):
        y = y * (1.5 - 0.5 * x * y * y)
    return y


def _make_atom_kernel(n_pad):
    nps = n_pad // NS
    npa = n_pad // NW

    @functools.partial(
        pl.kernel,
        out_type=jax.ShapeDtypeStruct((n_pad, 8), jnp.float32),
        mesh=plsc.VectorSubcoreMesh(core_axis_name="c", subcore_axis_name="s"),
        compiler_params=pltpu.CompilerParams(needs_layout_passes=False, use_tc_tiling_on_sc=False),
        scratch_types=[
            pltpu.VMEM((1024,), jnp.float32),
            pltpu.VMEM((nps,), jnp.int32),
            pltpu.VMEM((nps,), jnp.float32),
            pltpu.VMEM((96,), jnp.float32),
            pltpu.VMEM((96,), jnp.float32),
            pltpu.VMEM((8, 128), jnp.int32),
            pltpu.VMEM((1024,), jnp.float32),
            pltpu.VMEM((32,), jnp.float32),
            pltpu.VMEM((npa,), jnp.int32),
            pltpu.VMEM((npa,), jnp.float32),
            pltpu.VMEM((npa,), jnp.int32),
            pltpu.VMEM((npa, 8), jnp.float32),
            pltpu.VMEM_SHARED((1024,), jnp.float32),
        ],
    )
    def atom_kernel(charge_hbm, idxm_hbm, z_hbm, sqc6_hbm, sqr4_hbm, idx8_hbm,
                    tab_hbm, acc_v, idxm1_v, chg1_v, sqc6_v, sqr4_v, idx8_v,
                    binsf_v, qdiv_v, idxm2_v, chg2_v, z2_v, tab_v, bins_sh):
        c = lax.axis_index("c")
        s = lax.axis_index("s")
        wid = s * NC + c
        iota = lax.iota(jnp.int32, 16)
        zero16 = jnp.zeros((16,), jnp.float32)

        pltpu.sync_copy(idxm_hbm.at[pl.ds(s * nps, nps)], idxm1_v)
        pltpu.sync_copy(charge_hbm.at[pl.ds(s * nps, nps)], chg1_v)
        pltpu.sync_copy(idx8_hbm, idx8_v)
        pltpu.sync_copy(sqc6_hbm, sqc6_v)
        pltpu.sync_copy(sqr4_hbm, sqr4_v)

        def zero_body(i, _):
            acc_v[pl.ds(i * 16, 16)] = zero16
            return 0
        lax.fori_loop(0, 64, zero_body, 0)

        @pl.when(s == 0)
        def _():
            pltpu.sync_copy(acc_v, bins_sh)
        plsc.subcore_barrier()

        m_lo = idxm1_v[pl.ds(0, 16)][0]
        m_hi = idxm1_v[pl.ds(nps - 16, 16)][15]

        def m_body(m, _):
            def v_body(v, carry):
                qa_, ca_ = carry
                im = idxm1_v[pl.ds(v * 16, 16)]
                ch = chg1_v[pl.ds(v * 16, 16)]
                mask = im == m
                qa_ = qa_ + jnp.where(mask, -ch, 0.0)
                ca_ = ca_ + jnp.where(mask, 1.0, 0.0)
                return (qa_, ca_)
            qa_, ca_ = lax.fori_loop(0, nps // 16, v_body, (zero16, zero16))
            plsc.addupdate(acc_v.at[pl.ds(m * 16, 16)], qa_)
            plsc.addupdate(acc_v.at[pl.ds(512 + m * 16, 16)], ca_)
            return 0
        lax.fori_loop(m_lo, m_hi + 1, m_body, 0)

        for k in range(8):
            pltpu.sync_copy(acc_v.at[pl.ds(k * 128, 128)],
                            bins_sh.at[idx8_v.at[k]], add=True)
        plsc.subcore_barrier()

        pltpu.sync_copy(bins_sh, binsf_v)
        qlo = qhi = clo = chi = zero16
        for k in range(16):
            qlo = qlo + plsc.load_gather(binsf_v, [iota * 16 + k])
            qhi = qhi + plsc.load_gather(binsf_v, [(iota + 16) * 16 + k])
            clo = clo + plsc.load_gather(binsf_v, [512 + iota * 16 + k])
            chi = chi + plsc.load_gather(binsf_v, [512 + (iota + 16) * 16 + k])
        qdiv_v[pl.ds(0, 16)] = qlo / jnp.maximum(clo, 1.0)
        qdiv_v[pl.ds(16, 16)] = qhi / jnp.maximum(chi, 1.0)

        base = wid * npa
        pltpu.sync_copy(idxm_hbm.at[pl.ds(base, npa)], idxm2_v)
        pltpu.sync_copy(charge_hbm.at[pl.ds(base, npa)], chg2_v)
        pltpu.sync_copy(z_hbm.at[pl.ds(base, npa)], z2_v)

        def a_body(v, _):
            off = v * 16
            ch = chg2_v[pl.ds(off, 16)]
            zz = z2_v[pl.ds(off, 16)]
            mm = jnp.minimum(idxm2_v[pl.ds(off, 16)], 31)
            qa = ch + plsc.load_gather(qdiv_v, [mm])
            sc6 = (plsc.load_gather(sqc6_v, [zz])
                   * jnp.exp(1.5 * (1.0 - jnp.exp(2.0 * qa))))
            srz = plsc.load_gather(sqr4_v, [zz])
            row = iota + off
            plsc.store_scatter(tab_v, [row, jnp.full((16,), 0, jnp.int32)], qa)
            plsc.store_scatter(tab_v, [row, jnp.full((16,), 1, jnp.int32)], sc6)
            plsc.store_scatter(tab_v, [row, jnp.full((16,), 2, jnp.int32)], srz)
            plsc.store_scatter(tab_v, [row, jnp.full((16,), 3, jnp.int32)], zero16)
            return 0
        lax.fori_loop(0, npa // 16, a_body, 0)
        pltpu.sync_copy(tab_v, tab_hbm.at[pl.ds(base, npa), :])

    return atom_kernel


def _make_edge_kernel(n_pad, n_acc, e_total, ec):
    ecd = ec // 128
    nchunk = e_total // ec
    k_per_tile = (nchunk + NW - 1) // NW
    nsl = n_acc // NS

    @functools.partial(
        pl.kernel,
        out_type=jax.ShapeDtypeStruct((NC, n_acc), jnp.float32),
        mesh=plsc.VectorSubcoreMesh(core_axis_name="c", subcore_axis_name="s"),
        compiler_params=pltpu.CompilerParams(needs_layout_passes=False, use_tc_tiling_on_sc=False),
        scratch_types=[
            pltpu.VMEM((2, ec, 3), jnp.float32),
            pltpu.VMEM((4, ec), jnp.int32),
            pltpu.VMEM((4, ec), jnp.int32),
            pltpu.VMEM((2, ec, 8), jnp.float32),
            pltpu.VMEM((2, ec, 8), jnp.float32),
            pltpu.VMEM((2, ec), jnp.float32),
            pltpu.VMEM_SHARED((n_acc,), jnp.float32),
            pltpu.SemaphoreType.DMA,
            pltpu.SemaphoreType.DMA,
            pltpu.SemaphoreType.DMA,
            pltpu.SemaphoreType.DMA,
            pltpu.SemaphoreType.DMA,
            pltpu.SemaphoreType.DMA,
        ],
    )
    def edge_kernel(tab_hbm, rij_hbm, ii_hbm, jj_hbm, zeros_hbm, part_hbm,
                    rij_v, ii_v, jj_v, ri_v, rj_v, val_v, acc_sh,
                    isem0, isem1, gsem0, gsem1, ssem0, ssem1):
        c = lax.axis_index("c")
        s = lax.axis_index("s")
        wid = s * NC + c
        iota = lax.iota(jnp.int32, 16)
        col0 = jnp.full((16,), 0, jnp.int32)
        col1 = jnp.full((16,), 1, jnp.int32)
        col2 = jnp.full((16,), 2, jnp.int32)

        @pl.when(s == 0)
        def _():
            pltpu.sync_copy(zeros_hbm, acc_sh)
        plsc.subcore_barrier()

        def in_copies(x, isem):
            p, q = x % 2, x % 4
            return [
                pltpu.make_async_copy(rij_hbm.at[pl.ds(x * NW * ec + wid * ec, ec)], rij_v.at[p], isem),
                pltpu.make_async_copy(ii_hbm.at[pl.ds(x * NW * ec + wid * ec, ec)], ii_v.at[q], isem),
                pltpu.make_async_copy(jj_hbm.at[pl.ds(x * NW * ec + wid * ec, ec)], jj_v.at[q], isem),
            ]

        def gather_copies(x, gsem):
            p, q = x % 2, x % 4
            cps = []
            for r in range(ecd):
                cps.append(pltpu.make_async_copy(
                    tab_hbm.at[ii_v.at[q, pl.ds(r * 128, 128)]],
                    ri_v.at[p, pl.ds(r * 128, 128)], gsem))
                cps.append(pltpu.make_async_copy(
                    tab_hbm.at[jj_v.at[q, pl.ds(r * 128, 128)]],
                    rj_v.at[p, pl.ds(r * 128, 128)], gsem))
            return cps

        def scatter_copies(x, ssem):
            p, q = x % 2, x % 4
            return [pltpu.make_async_copy(
                val_v.at[p, pl.ds(r * 128, 128)],
                acc_sh.at[ii_v.at[q, pl.ds(r * 128, 128)]], ssem)
                for r in range(ecd)]

        def pipe_step(t, isem_t, gsem_t, ssem_t, isem_p, gsem_p):
            @pl.when((t >= 4) & ((t - 4) * NW + wid < nchunk))
            def _():
                for dcp in scatter_copies(t - 4, ssem_t):
                    dcp.wait()

            @pl.when((t >= 1) & ((t - 1) * NW + wid < nchunk))
            def _():
                for dcp in in_copies(t - 1, isem_p):
                    dcp.wait()
                for gcp in gather_copies(t - 1, gsem_p):
                    gcp.start()

            @pl.when((t >= 2) & ((t - 2) * NW + wid < nchunk))
            def _():
                for gcp in gather_copies(t - 2, gsem_t):
                    gcp.wait()
                _compute(t - 2)
                for scp in scatter_copies(t - 2, ssem_t):
                    scp.start(add=True)

            @pl.when((t < k_per_tile) & (t * NW + wid < nchunk))
            def _():
                for icp in in_copies(t, isem_t):
                    icp.start()

        def pipe_body(tt, _):
            t = tt * 2
            pipe_step(t, isem0, gsem0, ssem0, isem1, gsem1)
            pipe_step(t + 1, isem1, gsem1, ssem1, isem0, gsem0)
            return 0

        def _compute(x):
            p = x % 2
            rij_p = rij_v.at[p]
            ri_p = ri_v.at[p]
            rj_p = rj_v.at[p]

            def v_body(v, _):
                off = v * 16
                row = iota + off
                x = plsc.load_gather(rij_p, [row, col0])
                y = plsc.load_gather(rij_p, [row, col1])
                z = plsc.load_gather(rij_p, [row, col2])
                qi = plsc.load_gather(ri_p, [row, col0])
                ci = plsc.load_gather(ri_p, [row, col1])
                si = plsc.load_gather(ri_p, [row, col2])
                qj = plsc.load_gather(rj_p, [row, col0])
                cj = plsc.load_gather(rj_p, [row, col1])
                sj = plsc.load_gather(rj_p, [row, col2])

                r2 = x * x + y * y + z * z
                rs = _rsqrt(r2)
                d = r2 * rs
                r2s = r2 + 1.0
                rss = _rsqrt(r2s)
                dsh = r2s * rss
                t = (d - CUTON) * SW_INV
                poly = 1.0 + (t * t * t) * (-10.0 + t * (15.0 - 6.0 * t))
                f = jnp.where(t <= 0.0, 1.0,
                              jnp.where(t >= 1.0, 0.0, poly))
                coul = jnp.where(d < LR, rs + d * 0.01 - 0.2, 0.0)
                damp = jnp.where(dsh < LR, rss + dsh * 0.01 - 0.2, 0.0)
                epair = ((KE * 0.5) * qi * qj
                         * (f * damp + (1.0 - f) * coul))
                c6 = ci * cj
                sqq = SQRT3 * si * sj
                qq = sqq * sqq
                r0 = A1 * sqq + A2
                r02 = r0 * r0
                r06 = r02 * r02 * r02
                r08 = r06 * r02
                r6 = r2 * r2 * r2
                r8 = r6 * r2
                dpair = -0.5 * (S6 * c6 / (r6 + r06)
                                + S8 * c6 * qq / (r8 + r08))
                val_v[p, pl.ds(off, 16)] = epair + dpair
                return 0
            lax.fori_loop(0, ec // 16, v_body, 0)

        lax.fori_loop(0, (k_per_tile + 4 + 1) // 2, pipe_body, 0)
        plsc.subcore_barrier()

        pltpu.sync_copy(acc_sh.at[pl.ds(s * nsl, nsl)],
                        part_hbm.at[c, pl.ds(s * nsl, nsl)])

    return edge_kernel


def _combine_body(p_ref, e_ref, o_ref):
    o_ref[...] = p_ref[0] + p_ref[1] + e_ref[...]


def kernel(yi, Z, Rij, idx_i, idx_j, idx_m, c6_ref, r4r2):
    n = Z.shape[0]
    e_total = idx_i.shape[0]
    f32, i32 = jnp.float32, jnp.int32

    n_pad = ((n + 512 - 1) // 512) * 512
    n_acc = ((n + 128 - 1) // 128) * 128
    ec = 2048

    energy = yi[:, 0]
    charge = yi[:, 1]

    charge_p = jnp.concatenate([charge, jnp.zeros((n_pad - n,), f32)])
    idxm_p = jnp.concatenate([idx_m, jnp.full((n_pad - n,), 32, i32)])
    z_p = jnp.concatenate([Z, jnp.zeros((n_pad - n,), i32)])
    sqc6 = jnp.sqrt(jnp.concatenate([c6_ref, jnp.ones((1,), f32)]))
    sqr4 = jnp.sqrt(jnp.concatenate([r4r2, jnp.ones((1,), f32)]))
    idx8 = jnp.arange(1024, dtype=i32).reshape(8, 128)

    tab = _make_atom_kernel(n_pad)(charge_p, idxm_p, z_p, sqc6, sqr4, idx8)

    zeros_acc = jnp.zeros((n_acc,), f32)
    part = _make_edge_kernel(n_pad, n_acc, e_total, ec)(
        tab, Rij, idx_i, idx_j, zeros_acc)

    en_p = jnp.concatenate([energy, jnp.zeros((n_acc - n,), f32)])
    rows = n_acc // 128
    out2 = pl.pallas_call(
        _combine_body,
        out_shape=jax.ShapeDtypeStruct((rows, 128), f32),
    )(part.reshape(NC, rows, 128), en_p.reshape(rows, 128))
    return out2.reshape(n_acc)[:n][:, None]


# speedup vs baseline: 425.8511x; 12.4185x over previous
import functools

import jax
import jax.numpy as jnp
from jax import lax
from jax.experimental import pallas as pl
from jax.experimental.pallas import tpu as pltpu
from jax.experimental.pallas import tpu_sc as plsc

KE = 14.399645
S6, S8, A1, A2 = 1.0, 1.2, 0.4, 5.0
CUTON = 2.5
SW_INV = 0.2
LR = 10.0
SQRT3 = 1.7320508075688772

NC, NS, NW, L = 2, 16, 32, 16


def _rsqrt(x):
    b = lax.bitcast_convert_type(x, jnp.int32)
    y = lax.bitcast_convert_type(
        jnp.int32(0x5F3759DF) - lax.shift_right_arithmetic(b, 1), jnp.float32)
    for _ in range(3):
        y = y * (1.5 - 0.5 * x * y * y)
    return y


def _make_atom_kernel(n_pad):
    nps = n_pad // NS
    npa = n_pad // NW

    @functools.partial(
        pl.kernel,
        out_type=jax.ShapeDtypeStruct((n_pad, 8), jnp.float32),
        mesh=plsc.VectorSubcoreMesh(core_axis_name="c", subcore_axis_name="s"),
        compiler_params=pltpu.CompilerParams(needs_layout_passes=False, use_tc_tiling_on_sc=False),
        scratch_types=[
            pltpu.VMEM((1024,), jnp.float32),
            pltpu.VMEM((nps,), jnp.int32),
            pltpu.VMEM((nps,), jnp.float32),
            pltpu.VMEM((96,), jnp.float32),
            pltpu.VMEM((96,), jnp.float32),
            pltpu.VMEM((8, 128), jnp.int32),
            pltpu.VMEM((1024,), jnp.float32),
            pltpu.VMEM((32,), jnp.float32),
            pltpu.VMEM((npa,), jnp.int32),
            pltpu.VMEM((npa,), jnp.float32),
            pltpu.VMEM((npa,), jnp.int32),
            pltpu.VMEM((npa, 8), jnp.float32),
            pltpu.VMEM_SHARED((1024,), jnp.float32),
        ],
    )
    def atom_kernel(charge_hbm, idxm_hbm, z_hbm, sqc6_hbm, sqr4_hbm, idx8_hbm,
                    tab_hbm, acc_v, idxm1_v, chg1_v, sqc6_v, sqr4_v, idx8_v,
                    binsf_v, qdiv_v, idxm2_v, chg2_v, z2_v, tab_v, bins_sh):
        c = lax.axis_index("c")
        s = lax.axis_index("s")
        wid = s * NC + c
        iota = lax.iota(jnp.int32, 16)
        zero16 = jnp.zeros((16,), jnp.float32)

        pltpu.sync_copy(idxm_hbm.at[pl.ds(s * nps, nps)], idxm1_v)
        pltpu.sync_copy(charge_hbm.at[pl.ds(s * nps, nps)], chg1_v)
        pltpu.sync_copy(idx8_hbm, idx8_v)
        pltpu.sync_copy(sqc6_hbm, sqc6_v)
        pltpu.sync_copy(sqr4_hbm, sqr4_v)

        def zero_body(i, _):
            acc_v[pl.ds(i * 16, 16)] = zero16
            return 0
        lax.fori_loop(0, 64, zero_body, 0)

        @pl.when(s == 0)
        def _():
            pltpu.sync_copy(acc_v, bins_sh)
        plsc.subcore_barrier()

        m_lo = idxm1_v[pl.ds(0, 16)][0]
        m_hi = idxm1_v[pl.ds(nps - 16, 16)][15]

        def m_body(m, _):
            def v_body(v, carry):
                qa_, ca_ = carry
                im = idxm1_v[pl.ds(v * 16, 16)]
                ch = chg1_v[pl.ds(v * 16, 16)]
                mask = im == m
                qa_ = qa_ + jnp.where(mask, -ch, 0.0)
                ca_ = ca_ + jnp.where(mask, 1.0, 0.0)
                return (qa_, ca_)
            qa_, ca_ = lax.fori_loop(0, nps // 16, v_body, (zero16, zero16))
            plsc.addupdate(acc_v.at[pl.ds(m * 16, 16)], qa_)
            plsc.addupdate(acc_v.at[pl.ds(512 + m * 16, 16)], ca_)
            return 0
        lax.fori_loop(m_lo, m_hi + 1, m_body, 0)

        for k in range(8):
            pltpu.sync_copy(acc_v.at[pl.ds(k * 128, 128)],
                            bins_sh.at[idx8_v.at[k]], add=True)
        plsc.subcore_barrier()

        pltpu.sync_copy(bins_sh, binsf_v)
        qlo = qhi = clo = chi = zero16
        for k in range(16):
            qlo = qlo + plsc.load_gather(binsf_v, [iota * 16 + k])
            qhi = qhi + plsc.load_gather(binsf_v, [(iota + 16) * 16 + k])
            clo = clo + plsc.load_gather(binsf_v, [512 + iota * 16 + k])
            chi = chi + plsc.load_gather(binsf_v, [512 + (iota + 16) * 16 + k])
        qdiv_v[pl.ds(0, 16)] = qlo / jnp.maximum(clo, 1.0)
        qdiv_v[pl.ds(16, 16)] = qhi / jnp.maximum(chi, 1.0)

        base = wid * npa
        pltpu.sync_copy(idxm_hbm.at[pl.ds(base, npa)], idxm2_v)
        pltpu.sync_copy(charge_hbm.at[pl.ds(base, npa)], chg2_v)
        pltpu.sync_copy(z_hbm.at[pl.ds(base, npa)], z2_v)

        def a_body(v, _):
            off = v * 16
            ch = chg2_v[pl.ds(off, 16)]
            zz = z2_v[pl.ds(off, 16)]
            mm = jnp.minimum(idxm2_v[pl.ds(off, 16)], 31)
            qa = ch + plsc.load_gather(qdiv_v, [mm])
            sc6 = (plsc.load_gather(sqc6_v, [zz])
                   * jnp.exp(1.5 * (1.0 - jnp.exp(2.0 * qa))))
            srz = plsc.load_gather(sqr4_v, [zz])
            row = iota + off
            plsc.store_scatter(tab_v, [row, jnp.full((16,), 0, jnp.int32)], qa)
            plsc.store_scatter(tab_v, [row, jnp.full((16,), 1, jnp.int32)], sc6)
            plsc.store_scatter(tab_v, [row, jnp.full((16,), 2, jnp.int32)], srz)
            plsc.store_scatter(tab_v, [row, jnp.full((16,), 3, jnp.int32)], zero16)
            return 0
        lax.fori_loop(0, npa // 16, a_body, 0)
        pltpu.sync_copy(tab_v, tab_hbm.at[pl.ds(base, npa), :])

    return atom_kernel


def _make_edge_kernel(n_pad, n_acc, e_total, ec):
    ecd = ec // 128
    nchunk = e_total // ec
    k_per_tile = (nchunk + NW - 1) // NW
    nsl = n_acc // NS

    @functools.partial(
        pl.kernel,
        out_type=jax.ShapeDtypeStruct((NC, n_acc), jnp.float32),
        mesh=plsc.VectorSubcoreMesh(core_axis_name="c", subcore_axis_name="s"),
        compiler_params=pltpu.CompilerParams(needs_layout_passes=False, use_tc_tiling_on_sc=False),
        scratch_types=[
            pltpu.VMEM((2, ec), jnp.float32),
            pltpu.VMEM((2, ec), jnp.float32),
            pltpu.VMEM((2, ec), jnp.float32),
            pltpu.VMEM((4, ec), jnp.int32),
            pltpu.VMEM((4, ec), jnp.int32),
            pltpu.VMEM((2, ec, 8), jnp.float32),
            pltpu.VMEM((2, ec, 8), jnp.float32),
            pltpu.VMEM((2, ec), jnp.float32),
            pltpu.VMEM_SHARED((n_acc,), jnp.float32),
            pltpu.SemaphoreType.DMA,
            pltpu.SemaphoreType.DMA,
            pltpu.SemaphoreType.DMA,
            pltpu.SemaphoreType.DMA,
            pltpu.SemaphoreType.DMA,
            pltpu.SemaphoreType.DMA,
        ],
    )
    def edge_kernel(tab_hbm, xs_hbm, ys_hbm, zs_hbm, ii_hbm, jj_hbm,
                    zeros_hbm, part_hbm,
                    x_v, y_v, z_v, ii_v, jj_v, ri_v, rj_v, val_v, acc_sh,
                    isem0, isem1, gsem0, gsem1, ssem0, ssem1):
        c = lax.axis_index("c")
        s = lax.axis_index("s")
        wid = s * NC + c
        iota = lax.iota(jnp.int32, 16)
        col0 = jnp.full((16,), 0, jnp.int32)
        col1 = jnp.full((16,), 1, jnp.int32)
        col2 = jnp.full((16,), 2, jnp.int32)

        @pl.when(s == 0)
        def _():
            pltpu.sync_copy(zeros_hbm, acc_sh)
        plsc.subcore_barrier()

        def in_copies(x, isem):
            p, q = x % 2, x % 4
            eb = x * NW * ec + wid * ec
            return [
                pltpu.make_async_copy(xs_hbm.at[pl.ds(eb, ec)], x_v.at[p], isem),
                pltpu.make_async_copy(ys_hbm.at[pl.ds(eb, ec)], y_v.at[p], isem),
                pltpu.make_async_copy(zs_hbm.at[pl.ds(eb, ec)], z_v.at[p], isem),
                pltpu.make_async_copy(ii_hbm.at[pl.ds(eb, ec)], ii_v.at[q], isem),
                pltpu.make_async_copy(jj_hbm.at[pl.ds(eb, ec)], jj_v.at[q], isem),
            ]

        def gather_copies(x, gsem):
            p, q = x % 2, x % 4
            cps = []
            for r in range(ecd):
                cps.append(pltpu.make_async_copy(
                    tab_hbm.at[ii_v.at[q, pl.ds(r * 128, 128)]],
                    ri_v.at[p, pl.ds(r * 128, 128)], gsem))
                cps.append(pltpu.make_async_copy(
                    tab_hbm.at[jj_v.at[q, pl.ds(r * 128, 128)]],
                    rj_v.at[p, pl.ds(r * 128, 128)], gsem))
            return cps

        def scatter_copies(x, ssem):
            p, q = x % 2, x % 4
            return [pltpu.make_async_copy(
                val_v.at[p, pl.ds(r * 128, 128)],
                acc_sh.at[ii_v.at[q, pl.ds(r * 128, 128)]], ssem)
                for r in range(ecd)]

        def pipe_step(t, isem_t, gsem_t, ssem_t, isem_p, gsem_p):
            @pl.when((t >= 4) & ((t - 4) * NW + wid < nchunk))
            def _():
                for dcp in scatter_copies(t - 4, ssem_t):
                    dcp.wait()

            @pl.when((t >= 1) & ((t - 1) * NW + wid < nchunk))
            def _():
                for dcp in in_copies(t - 1, isem_p):
                    dcp.wait()
                for gcp in gather_copies(t - 1, gsem_p):
                    gcp.start()

            @pl.when((t >= 2) & ((t - 2) * NW + wid < nchunk))
            def _():
                for gcp in gather_copies(t - 2, gsem_t):
                    gcp.wait()
                _compute(t - 2)
                for scp in scatter_copies(t - 2, ssem_t):
                    scp.start(add=True)

            @pl.when((t < k_per_tile) & (t * NW + wid < nchunk))
            def _():
                for icp in in_copies(t, isem_t):
                    icp.start()

        def pipe_body(tt, _):
            t = tt * 2
            pipe_step(t, isem0, gsem0, ssem0, isem1, gsem1)
            pipe_step(t + 1, isem1, gsem1, ssem1, isem0, gsem0)
            return 0

        def _compute(x):
            p = x % 2
            ri_p = ri_v.at[p]
            rj_p = rj_v.at[p]

            def v_body(v, _):
                off = v * 16
                row = iota + off
                x = x_v[p, pl.ds(off, 16)]
                y = y_v[p, pl.ds(off, 16)]
                z = z_v[p, pl.ds(off, 16)]
                qi = plsc.load_gather(ri_p, [row, col0])
                ci = plsc.load_gather(ri_p, [row, col1])
                si = plsc.load_gather(ri_p, [row, col2])
                qj = plsc.load_gather(rj_p, [row, col0])
                cj = plsc.load_gather(rj_p, [row, col1])
                sj = plsc.load_gather(rj_p, [row, col2])

                r2 = x * x + y * y + z * z
                rs = _rsqrt(r2)
                d = r2 * rs
                r2s = r2 + 1.0
                rss = _rsqrt(r2s)
                dsh = r2s * rss
                t = (d - CUTON) * SW_INV
                poly = 1.0 + (t * t * t) * (-10.0 + t * (15.0 - 6.0 * t))
                f = jnp.where(t <= 0.0, 1.0,
                              jnp.where(t >= 1.0, 0.0, poly))
                coul = jnp.where(d < LR, rs + d * 0.01 - 0.2, 0.0)
                damp = jnp.where(dsh < LR, rss + dsh * 0.01 - 0.2, 0.0)
                epair = ((KE * 0.5) * qi * qj
                         * (f * damp + (1.0 - f) * coul))
                c6 = ci * cj
                sqq = SQRT3 * si * sj
                qq = sqq * sqq
                r0 = A1 * sqq + A2
                r02 = r0 * r0
                r06 = r02 * r02 * r02
                r08 = r06 * r02
                r6 = r2 * r2 * r2
                r8 = r6 * r2
                dpair = -0.5 * (S6 * c6 / (r6 + r06)
                                + S8 * c6 * qq / (r8 + r08))
                val_v[p, pl.ds(off, 16)] = epair + dpair
                return 0
            lax.fori_loop(0, ec // 16, v_body, 0)

        lax.fori_loop(0, (k_per_tile + 4 + 1) // 2, pipe_body, 0)
        plsc.subcore_barrier()

        pltpu.sync_copy(acc_sh.at[pl.ds(s * nsl, nsl)],
                        part_hbm.at[c, pl.ds(s * nsl, nsl)])

    return edge_kernel


def _combine_body(p_ref, e_ref, o_ref):
    o_ref[...] = p_ref[0] + p_ref[1] + e_ref[...]


def kernel(yi, Z, Rij, idx_i, idx_j, idx_m, c6_ref, r4r2):
    n = Z.shape[0]
    e_total = idx_i.shape[0]
    f32, i32 = jnp.float32, jnp.int32

    n_pad = ((n + 512 - 1) // 512) * 512
    n_acc = ((n + 128 - 1) // 128) * 128
    ec = 2048

    energy = yi[:, 0]
    charge = yi[:, 1]

    charge_p = jnp.concatenate([charge, jnp.zeros((n_pad - n,), f32)])
    idxm_p = jnp.concatenate([idx_m, jnp.full((n_pad - n,), 32, i32)])
    z_p = jnp.concatenate([Z, jnp.zeros((n_pad - n,), i32)])
    sqc6 = jnp.sqrt(jnp.concatenate([c6_ref, jnp.ones((1,), f32)]))
    sqr4 = jnp.sqrt(jnp.concatenate([r4r2, jnp.ones((1,), f32)]))
    idx8 = jnp.arange(1024, dtype=i32).reshape(8, 128)

    tab = _make_atom_kernel(n_pad)(charge_p, idxm_p, z_p, sqc6, sqr4, idx8)

    zeros_acc = jnp.zeros((n_acc,), f32)
    part = _make_edge_kernel(n_pad, n_acc, e_total, ec)(
        tab, Rij[:, 0], Rij[:, 1], Rij[:, 2], idx_i, idx_j, zeros_acc)

    en_p = jnp.concatenate([energy, jnp.zeros((n_acc - n,), f32)])
    rows = n_acc // 128
    out2 = pl.pallas_call(
        _combine_body,
        out_shape=jax.ShapeDtypeStruct((rows, 128), f32),
    )(part.reshape(NC, rows, 128), en_p.reshape(rows, 128))
    return out2.reshape(n_acc)[:n][:, None]
